# baseline (speedup 1.0000x reference)
import jax
import jax.numpy as jnp
from jax import lax
from jax.experimental import pallas as pl
from jax.experimental.pallas import tpu as pltpu
from jax.experimental.pallas import tpu_sc as plsc

_D = 2048
_E = 12000
_NC = 2
_NS = 16
_NW = _NC * _NS
_EPT = _E // _NW
_B = 8
_NB = 48
_EPAD = _NB * _B
_LN = 16
_LB = 32
_CH = _D // _LB
_HD = _D // 2


_RING = 4


def _edge_score_body(x_hbm, rel_hbm, idx_hbm, out_hbm,
                     idx_v, rel_v, rows_v, scores_v, *sems):
    wid = lax.axis_index("s") * _NC + lax.axis_index("c")
    pltpu.sync_copy(rel_hbm, rel_v)

    for seg in range(3):
        rrow = 0 if seg == 0 else 1
        pltpu.sync_copy(idx_hbm.at[seg, wid], idx_v)

        for r in range(_RING):
            pltpu.async_copy(x_hbm.at[idx_v.at[r]], rows_v.at[r], sems[r])

        lane = jnp.arange(_LN, dtype=jnp.int32)

        @pl.loop(0, _NB, step=_RING)
        def _batches(b):
            svecs = [jnp.zeros((_LN,), jnp.float32)
                     for _ in range(_RING // 2)]
            for rbuf in range(_RING):
                bb = b + rbuf
                sem = sems[rbuf]
                pltpu.make_async_copy(
                    x_hbm.at[idx_v.at[bb]], rows_v.at[rbuf], sem).wait()
                buf = rows_v.at[rbuf]

                def _chunk(c, accs):
                    rw = plsc.bitcast(
                        rel_v[rrow, pl.ds(c * _LN, _LN)], jnp.bfloat16)
                    out = []
                    for e in range(_B):
                        h = plsc.bitcast(
                            buf[e, pl.ds(c * _LN, _LN)], jnp.bfloat16)
                        t = plsc.bitcast(
                            buf[e + _B, pl.ds(c * _LN, _LN)], jnp.bfloat16)
                        pe, po = plsc.unpack(
                            (h * t) * rw, format=plsc.PackFormat.INTERLEAVED)
                        a1, a2 = accs[2 * e], accs[2 * e + 1]
                        out.append(a1 + pe)
                        out.append(a2 + po)
                    return tuple(out)

                accs = lax.fori_loop(
                    0, _CH, _chunk,
                    tuple(jnp.zeros((_LN,), jnp.float32)
                          for _ in range(2 * _B)))
                for e in range(_B):
                    s = jnp.sum(accs[2 * e] + accs[2 * e + 1], axis=0)
                    s = jnp.clip(s, 0.0, 1.0)
                    svecs[rbuf // 2] = jnp.where(
                        lane == (rbuf % 2) * _B + e, s, svecs[rbuf // 2])

                @pl.when(bb + _RING < _NB)
                def _():
                    pltpu.async_copy(
                        x_hbm.at[idx_v.at[bb + _RING]], rows_v.at[rbuf], sem)

            for v in range(_RING // 2):
                scores_v[pl.ds((b + 2 * v) * _B, _LN)] = svecs[v]

        pltpu.sync_copy(scores_v, out_hbm.at[seg, wid])


_edge_score_sc = pl.kernel(
    _edge_score_body,
    out_type=jax.ShapeDtypeStruct((3, _NW, _EPAD), jnp.float32),
    mesh=plsc.VectorSubcoreMesh(core_axis_name="c", subcore_axis_name="s"),
    compiler_params=pltpu.CompilerParams(needs_layout_passes=False),
    scratch_types=[
        pltpu.VMEM((_NB, 2 * _B), jnp.int32),
        pltpu.VMEM((2, _HD), jnp.int32),
        pltpu.VMEM((_RING, 2 * _B, _HD), jnp.int32),
        pltpu.VMEM((_EPAD,), jnp.float32),
    ] + [pltpu.SemaphoreType.DMA] * _RING,
)


def _pack_indices(edge_index):
    h = jnp.pad(edge_index[0].reshape(_NW, _EPT), ((0, 0), (0, _EPAD - _EPT)))
    t = jnp.pad(edge_index[1].reshape(_NW, _EPT), ((0, 0), (0, _EPAD - _EPT)))
    return jnp.concatenate(
        [h.reshape(_NW, _NB, _B), t.reshape(_NW, _NB, _B)], axis=-1)


_BM = 200


def _pack_rows_body(x_ref, out_ref):
    lo = jax.lax.bitcast_convert_type(x_ref[:, :_HD], jnp.uint32)
    hi = jax.lax.bitcast_convert_type(x_ref[:, _HD:], jnp.uint32)
    half = jnp.uint32(0x8000)
    word = ((lo + half) >> 16) | ((hi + half) & jnp.uint32(0xFFFF0000))
    out_ref[...] = jax.lax.bitcast_convert_type(word, jnp.int32)


def _pack_rows(x):
    n = x.shape[0]
    return pl.pallas_call(
        _pack_rows_body,
        grid=(n // _BM,),
        in_specs=[pl.BlockSpec((_BM, _D), lambda i: (i, 0))],
        out_specs=pl.BlockSpec((_BM, _HD), lambda i: (i, 0)),
        out_shape=jax.ShapeDtypeStruct((n, _HD), jnp.int32),
    )(x)


def _pack_rel(rel):
    b = jax.lax.bitcast_convert_type(rel, jnp.uint32) + jnp.uint32(0x8000)
    word = (b[:_HD] >> 16) | (b[_HD:] & jnp.uint32(0xFFFF0000))
    return jax.lax.bitcast_convert_type(word, jnp.int32)


def kernel(x, rel_ddi, rel_dpi, edge_index_ddi, edge_index_dpi,
           edge_index_ppi):
    idx = jnp.stack([_pack_indices(edge_index_ddi),
                     _pack_indices(edge_index_dpi),
                     _pack_indices(edge_index_ppi)])
    rel = jnp.stack([_pack_rel(rel_ddi), _pack_rel(rel_dpi)])
    out = _edge_score_sc(_pack_rows(x), rel, idx)
    return out[:, :, :_EPT].reshape(-1)


# speedup vs baseline: 1.0673x; 1.0673x over previous
import jax
import jax.numpy as jnp
from jax import lax
from jax.experimental import pallas as pl
from jax.experimental.pallas import tpu as pltpu
from jax.experimental.pallas import tpu_sc as plsc

_D = 2048
_E = 12000
_NC = 2
_NS = 16
_NW = _NC * _NS
_EPT = _E // _NW
_B = 8
_NB = 48
_EPAD = _NB * _B
_LN = 16
_LB = 32
_CH = _D // _LB
_HD = _D // 2


_RING = 4


def _edge_score_body(x_hbm, rel_hbm, idx_hbm, out_hbm,
                     idx_v, rel_v, rows_v, scores_v, *sems):
    wid = lax.axis_index("s") * _NC + lax.axis_index("c")
    pltpu.sync_copy(rel_hbm, rel_v)

    for seg in range(3):
        rrow = 0 if seg == 0 else 1
        pltpu.sync_copy(idx_hbm.at[seg, wid], idx_v)

        for r in range(_RING):
            pltpu.async_copy(x_hbm.at[idx_v.at[r]], rows_v.at[r], sems[r])

        lane = jnp.arange(_LN, dtype=jnp.int32)

        @pl.loop(0, _NB, step=_RING)
        def _batches(b):
            svecs = [jnp.zeros((_LN,), jnp.float32)
                     for _ in range(_RING // 2)]
            for rbuf in range(_RING):
                bb = b + rbuf
                sem = sems[rbuf]
                pltpu.make_async_copy(
                    x_hbm.at[idx_v.at[bb]], rows_v.at[rbuf], sem).wait()
                buf = rows_v.at[rbuf]

                def _chunk(c, accs):
                    rw = plsc.bitcast(
                        rel_v[rrow, pl.ds(c * _LN, _LN)], jnp.bfloat16)
                    out = []
                    for e in range(_B):
                        h = plsc.bitcast(
                            buf[e, pl.ds(c * _LN, _LN)], jnp.bfloat16)
                        t = plsc.bitcast(
                            buf[e + _B, pl.ds(c * _LN, _LN)], jnp.bfloat16)
                        pe, po = plsc.unpack(
                            (h * t) * rw, format=plsc.PackFormat.INTERLEAVED)
                        a1, a2 = accs[2 * e], accs[2 * e + 1]
                        out.append(a1 + pe)
                        out.append(a2 + po)
                    return tuple(out)

                accs = lax.fori_loop(
                    0, _CH, _chunk,
                    tuple(jnp.zeros((_LN,), jnp.float32)
                          for _ in range(2 * _B)))
                for e in range(_B):
                    s = jnp.sum(accs[2 * e] + accs[2 * e + 1], axis=0)
                    s = jnp.clip(s, 0.0, 1.0)
                    svecs[rbuf // 2] = jnp.where(
                        lane == (rbuf % 2) * _B + e, s, svecs[rbuf // 2])

                @pl.when(bb + _RING < _NB)
                def _():
                    pltpu.async_copy(
                        x_hbm.at[idx_v.at[bb + _RING]], rows_v.at[rbuf], sem)

            for v in range(_RING // 2):
                scores_v[pl.ds((b + 2 * v) * _B, _LN)] = svecs[v]

        pltpu.sync_copy(scores_v, out_hbm.at[seg, wid])


_edge_score_sc = pl.kernel(
    _edge_score_body,
    out_type=jax.ShapeDtypeStruct((3, _NW, _EPAD), jnp.float32),
    mesh=plsc.VectorSubcoreMesh(core_axis_name="c", subcore_axis_name="s"),
    compiler_params=pltpu.CompilerParams(needs_layout_passes=False),
    scratch_types=[
        pltpu.VMEM((_NB, 2 * _B), jnp.int32),
        pltpu.VMEM((2, _HD), jnp.int32),
        pltpu.VMEM((_RING, 2 * _B, _HD), jnp.int32),
        pltpu.VMEM((_EPAD,), jnp.float32),
    ] + [pltpu.SemaphoreType.DMA] * _RING,
)


def _pack_indices(edge_index):
    h = jnp.pad(edge_index[0].reshape(_NW, _EPT), ((0, 0), (0, _EPAD - _EPT)))
    t = jnp.pad(edge_index[1].reshape(_NW, _EPT), ((0, 0), (0, _EPAD - _EPT)))
    return jnp.concatenate(
        [h.reshape(_NW, _NB, _B), t.reshape(_NW, _NB, _B)], axis=-1)


_BM = 1000


def _pack_rows_body(x_ref, out_ref):
    lo = jax.lax.bitcast_convert_type(x_ref[:, :_HD], jnp.uint32)
    hi = jax.lax.bitcast_convert_type(x_ref[:, _HD:], jnp.uint32)
    half = jnp.uint32(0x8000)
    word = ((lo + half) >> 16) | ((hi + half) & jnp.uint32(0xFFFF0000))
    out_ref[...] = jax.lax.bitcast_convert_type(word, jnp.int32)


def _pack_rows(x):
    n = x.shape[0]
    return pl.pallas_call(
        _pack_rows_body,
        grid=(n // _BM,),
        in_specs=[pl.BlockSpec((_BM, _D), lambda i: (i, 0))],
        out_specs=pl.BlockSpec((_BM, _HD), lambda i: (i, 0)),
        out_shape=jax.ShapeDtypeStruct((n, _HD), jnp.int32),
    )(x)


def _pack_rel(rel):
    b = jax.lax.bitcast_convert_type(rel, jnp.uint32) + jnp.uint32(0x8000)
    word = (b[:_HD] >> 16) | (b[_HD:] & jnp.uint32(0xFFFF0000))
    return jax.lax.bitcast_convert_type(word, jnp.int32)


def kernel(x, rel_ddi, rel_dpi, edge_index_ddi, edge_index_dpi,
           edge_index_ppi):
    idx = jnp.stack([_pack_indices(edge_index_ddi),
                     _pack_indices(edge_index_dpi),
                     _pack_indices(edge_index_ppi)])
    rel = jnp.stack([_pack_rel(rel_ddi), _pack_rel(rel_dpi)])
    out = _edge_score_sc(_pack_rows(x), rel, idx)
    return out[:, :, :_EPT].reshape(-1)
